# Initial kernel scaffold; baseline (speedup 1.0000x reference)
import jax
import jax.numpy as jnp
from jax import lax
from jax.experimental import pallas as pl
from jax.experimental.pallas import tpu as pltpu
from jax.experimental.pallas import tpu_sc as plsc

_NC = 2
_NS = 16
_NW = _NC * _NS
_CH = 32
_NB = 4


def _sc_gather(table, idx_flat, B, D):
    bpw = B // _NW
    nch = bpw // _CH
    ngrp = nch // _NB
    assert bpw * _NW == B and nch * _CH == bpw and ngrp * _NB == nch

    mesh = plsc.VectorSubcoreMesh(core_axis_name="c", subcore_axis_name="s")

    def body(table_h, idx_h, out_h, idx_v, rows_v, *sems):
        gsem = sems[:_NB]
        ssem = sems[_NB:]
        wid = lax.axis_index("s") * _NC + lax.axis_index("c")
        base = wid * bpw
        pltpu.sync_copy(idx_h.at[pl.ds(base, bpw)], idx_v)

        def g_copy(t, b):
            return pltpu.make_async_copy(
                table_h.at[idx_v.at[pl.ds(t * _CH, _CH)]],
                rows_v.at[b], gsem[b])

        def s_copy(t, b):
            return pltpu.make_async_copy(
                rows_v.at[b],
                out_h.at[pl.ds(base + t * _CH, _CH)], ssem[b])

        g_copy(0, 0).start()
        g_copy(1, 1).start()

        def step(t, b, wait_sc, start_g):
            g_copy(t, b).wait()
            s_copy(t, b).start()
            if start_g:
                u = t + 2
                bu = (b + 2) % _NB
                if wait_sc:
                    s_copy(u - _NB, bu).wait()
                g_copy(u, bu).start()

        step(0, 0, False, True)
        step(1, 1, False, True)
        step(2, 2, True, True)
        step(3, 3, True, True)

        def group(grp, carry):
            t0 = grp * _NB
            for b in range(_NB):
                step(t0 + b, b, True, True)
            return carry

        lax.fori_loop(1, ngrp - 1, group, 0)

        t0 = (ngrp - 1) * _NB
        step(t0 + 0, 0, True, True)
        step(t0 + 1, 1, True, True)
        step(t0 + 2, 2, False, False)
        step(t0 + 3, 3, False, False)

        for b in range(_NB):
            s_copy(nch - _NB + b, b).wait()

    call = pl.kernel(
        body,
        out_type=jax.ShapeDtypeStruct((B, D), jnp.float32),
        mesh=mesh,
        scratch_types=[
            pltpu.VMEM((bpw,), jnp.int32),
            pltpu.VMEM((_NB, _CH, D), jnp.float32),
        ] + [pltpu.SemaphoreType.DMA] * (2 * _NB),
    )
    return call(table, idx_flat)


def kernel(idx, targets, table):
    B, L = idx.shape
    V, D = table.shape
    idx_flat = idx.reshape(B * L)
    out = _sc_gather(table, idx_flat, B * L, D)
    return out.reshape(B, L, D)


# speedup vs baseline: 1.4385x; 1.4385x over previous
import jax
import jax.numpy as jnp
from jax import lax
from jax.experimental import pallas as pl
from jax.experimental.pallas import tpu as pltpu
from jax.experimental.pallas import tpu_sc as plsc

_NC = 2
_NS = 16
_NW = _NC * _NS
_CH = 32
_NB = 4


def _sc_gather(table, idx_flat, B, D):
    bpw = B // _NW
    nch = bpw // _CH
    ngrp = nch // _NB
    assert bpw * _NW == B and nch * _CH == bpw and ngrp * _NB == nch

    mesh = plsc.VectorSubcoreMesh(core_axis_name="c", subcore_axis_name="s")

    def body(table_h, idx_h, out_h, idx_v, rows_v, *sems):
        gsem = sems[:_NB]
        ssem = sems[_NB:]
        wid = lax.axis_index("s") * _NC + lax.axis_index("c")
        base = wid * bpw
        pltpu.sync_copy(idx_h.at[pl.ds(base, bpw)], idx_v)

        def g_copy(t, b):
            return pltpu.make_async_copy(
                table_h.at[idx_v.at[pl.ds(t * _CH, _CH)]],
                rows_v.at[b], gsem[b])

        def s_copy(t, b):
            return pltpu.make_async_copy(
                rows_v.at[b],
                out_h.at[pl.ds(base + t * _CH, _CH)], ssem[b])

        g_copy(0, 0).start()
        g_copy(1, 1).start()

        def step(t, b, wait_sc, start_g):
            g_copy(t, b).wait()
            s_copy(t, b).start()
            if start_g:
                u = t + 2
                bu = (b + 2) % _NB
                if wait_sc:
                    s_copy(u - _NB, bu).wait()
                g_copy(u, bu).start()

        step(0, 0, False, True)
        step(1, 1, False, True)
        step(2, 2, True, True)
        step(3, 3, True, True)

        def group(grp, carry):
            t0 = grp * _NB
            for b in range(_NB):
                step(t0 + b, b, True, True)
            return carry

        lax.fori_loop(1, ngrp - 1, group, 0)

        t0 = (ngrp - 1) * _NB
        step(t0 + 0, 0, True, True)
        step(t0 + 1, 1, True, True)
        step(t0 + 2, 2, False, False)
        step(t0 + 3, 3, False, False)

        for b in range(_NB):
            s_copy(nch - _NB + b, b).wait()

    call = pl.kernel(
        body,
        out_type=jax.ShapeDtypeStruct((B, D), jnp.float32),
        mesh=mesh,
        scratch_types=[
            pltpu.VMEM((bpw,), jnp.int32),
            pltpu.VMEM((_NB, _CH, D), jnp.float32),
        ] + [pltpu.SemaphoreType.DMA] * (2 * _NB),
        compiler_params=pltpu.CompilerParams(use_tc_tiling_on_sc=False),
    )
    return call(table, idx_flat)


def kernel(idx, targets, table):
    B, L = idx.shape
    V, D = table.shape
    idx_flat = idx.reshape(B * L)
    out = _sc_gather(table, idx_flat, B * L, D)
    return out.reshape(B, L, D)
